# baseline (speedup 1.0000x reference)
import functools

import jax
import jax.numpy as jnp
from jax import lax
from jax.experimental import pallas as pl
from jax.experimental.pallas import tpu as pltpu
from jax.experimental.pallas import tpu_sc as plsc

V = 1000
BT = 4096 * 20
NC = 2
NS = 16
L = 16
NW = NC * NS
B_PER_W = BT // NW

TR = V // 8
TI = BT // 128
CT = 32
NCH = TI // CT
SLABS = 4


def _pre_body(t_ref, lse_ref, tt_ref):
    t = t_ref[...]
    m = jnp.max(t, axis=1, keepdims=True)
    s = jnp.sum(jnp.exp(t - m), axis=1, keepdims=True)
    lse_ref[...] = m + jnp.log(s)
    tt_ref[...] = t.T


_pre_call = pl.pallas_call(
    _pre_body,
    out_shape=(
        jax.ShapeDtypeStruct((V, 1), jnp.float32),
        jax.ShapeDtypeStruct((V, V), jnp.float32),
    ),
)


def _fin_body(p_ref, o_ref):
    o_ref[...] = (jnp.sum(p_ref[...]) * (1.0 / BT)).reshape(1, 1)


_fin_call = pl.pallas_call(
    _fin_body,
    out_shape=jax.ShapeDtypeStruct((1, 1), jnp.float32),
)


_mesh = plsc.VectorSubcoreMesh(core_axis_name="c", subcore_axis_name="s")


@functools.partial(
    pl.kernel,
    out_type=(
        jax.ShapeDtypeStruct((TR, TI, 8, 128), jnp.float32),
        jax.ShapeDtypeStruct((NW, L), jnp.float32),
    ),
    mesh=_mesh,
    compiler_params=pltpu.CompilerParams(
        use_tc_tiling_on_sc=False, needs_layout_passes=False),
    scratch_types=[
        pltpu.VMEM((8 * V,), jnp.float32),
        pltpu.VMEM((CT * 128,), jnp.int32),
        pltpu.VMEM((CT, 8, 128), jnp.float32),
        pltpu.VMEM((CT, 8, 128), jnp.float32),
        pltpu.VMEM((B_PER_W,), jnp.int32),
        pltpu.VMEM((B_PER_W,), jnp.int32),
        pltpu.VMEM((B_PER_W,), jnp.int32),
        pltpu.VMEM((B_PER_W,), jnp.float32),
        pltpu.VMEM((V,), jnp.float32),
        pltpu.VMEM((L,), jnp.float32),
        pltpu.SemaphoreType.DMA,
        pltpu.SemaphoreType.DMA,
        pltpu.SemaphoreType.DMA,
    ],
)
def _sc_gather(x_hbm, tgt_hbm, lse_hbm, tt_hbm, out_hbm, part_hbm,
               slab, xc, buf0, buf1, xloc, tloc, fidx, picked, lseloc,
               accv, sem0, sem1, semg):
    wid = lax.axis_index("s") * NC + lax.axis_index("c")

    base = wid * B_PER_W
    pltpu.sync_copy(x_hbm.at[pl.ds(base, B_PER_W)], xloc)
    pltpu.sync_copy(tgt_hbm.at[pl.ds(base, B_PER_W)], tloc)
    pltpu.sync_copy(lse_hbm, lseloc)

    def fidx_body(k, carry):
        xv = xloc[pl.ds(k * L, L)]
        tv = tloc[pl.ds(k * L, L)]
        fidx[pl.ds(k * L, L)] = tv * V + xv
        return carry

    lax.fori_loop(0, B_PER_W // L, fidx_body, jnp.int32(0))
    gather_handle = pltpu.async_copy(tt_hbm.at[fidx], picked, semg)

    def fill(buf, c):
        pltpu.sync_copy(x_hbm.at[pl.ds(c * (CT * 128), CT * 128)], xc)

        def tile_body(t, carry):
            for g in range(8):
                xg = xc[pl.ds(t * 128 + g * 16, L)]
                for v in range(8):
                    val = plsc.load_gather(slab, [xg + (v * V)])
                    buf[t, v, pl.ds(g * 16, L)] = val
            return carry

        lax.fori_loop(0, CT, tile_body, jnp.int32(0))

    for s in range(SLABS):
        tr = wid + s * NW

        @pl.when(tr < TR)
        def _slab_work():
            pltpu.sync_copy(tt_hbm.at[pl.ds(tr * (8 * V), 8 * V)], slab)

            def chunk_pair(cc, carry):
                c0 = cc * 2
                for half, buf, sem in ((0, buf0, sem0), (1, buf1, sem1)):
                    c = c0 + half

                    @pl.when(cc > 0)
                    def _drain():
                        pltpu.make_async_copy(
                            buf, out_hbm.at[tr, pl.ds((c - 2) * CT, CT)],
                            sem).wait()

                    fill(buf, c)
                    pltpu.async_copy(
                        buf, out_hbm.at[tr, pl.ds(c * CT, CT)], sem)
                return carry

            lax.fori_loop(0, NCH // 2, chunk_pair, jnp.int32(0))
            pltpu.make_async_copy(
                buf0, out_hbm.at[tr, pl.ds((NCH - 2) * CT, CT)], sem0).wait()
            pltpu.make_async_copy(
                buf1, out_hbm.at[tr, pl.ds((NCH - 1) * CT, CT)], sem1).wait()

    gather_handle.wait()

    def acc_body(k, acc):
        lsev = plsc.load_gather(lseloc, [xloc[pl.ds(k * L, L)]])
        return acc + (lsev - picked[pl.ds(k * L, L)])

    acc = lax.fori_loop(0, B_PER_W // L, acc_body,
                        jnp.zeros((L,), jnp.float32))
    accv[...] = acc
    pltpu.sync_copy(accv, part_hbm.at[wid])


def kernel(x, targets, table):
    xf = x.reshape(BT).astype(jnp.int32)
    tf = targets.reshape(BT).astype(jnp.int32)
    lse2, tt = _pre_call(table)
    out4, part = _sc_gather(xf, tf, lse2.reshape(V), tt.reshape(V * V))
    logits = out4.transpose(1, 3, 0, 2).reshape(BT, V)
    loss = _fin_call(part)[0, 0]
    return logits, loss


# speedup vs baseline: 2.6738x; 2.6738x over previous
import functools

import jax
import jax.numpy as jnp
from jax import lax
from jax.experimental import pallas as pl
from jax.experimental.pallas import tpu as pltpu
from jax.experimental.pallas import tpu_sc as plsc

V = 1000
BT = 4096 * 20
NC = 2
NS = 16
L = 16
NW = NC * NS
B_PER_W = BT // NW

TR = V // 8
TI = BT // 128
CT = 32
NCH = TI // CT
SLABS = 4


def _pre_body(t_ref, lse_ref, tt_ref):
    t = t_ref[...]
    m = jnp.max(t, axis=1, keepdims=True)
    s = jnp.sum(jnp.exp(t - m), axis=1, keepdims=True)
    lse_ref[...] = m + jnp.log(s)
    tt_ref[...] = t.T


_pre_call = pl.pallas_call(
    _pre_body,
    out_shape=(
        jax.ShapeDtypeStruct((V, 1), jnp.float32),
        jax.ShapeDtypeStruct((V, V), jnp.float32),
    ),
)


def _fin_body(p_ref, o_ref):
    o_ref[...] = (jnp.sum(p_ref[...]) * (1.0 / BT)).reshape(1, 1)


_fin_call = pl.pallas_call(
    _fin_body,
    out_shape=jax.ShapeDtypeStruct((1, 1), jnp.float32),
)


_mesh = plsc.VectorSubcoreMesh(core_axis_name="c", subcore_axis_name="s")


@functools.partial(
    pl.kernel,
    out_type=(
        jax.ShapeDtypeStruct((TR, TI, 8, 128), jnp.float32),
        jax.ShapeDtypeStruct((NW, L), jnp.float32),
    ),
    mesh=_mesh,
    compiler_params=pltpu.CompilerParams(
        use_tc_tiling_on_sc=False, needs_layout_passes=False),
    scratch_types=[
        pltpu.VMEM((8 * V,), jnp.float32),
        pltpu.VMEM((CT * 128,), jnp.int32),
        pltpu.VMEM((CT, 8, 128), jnp.float32),
        pltpu.VMEM((CT, 8, 128), jnp.float32),
        pltpu.VMEM((B_PER_W,), jnp.int32),
        pltpu.VMEM((B_PER_W,), jnp.int32),
        pltpu.VMEM((B_PER_W,), jnp.int32),
        pltpu.VMEM((B_PER_W,), jnp.float32),
        pltpu.VMEM((V,), jnp.float32),
        pltpu.VMEM((L,), jnp.float32),
        pltpu.SemaphoreType.DMA,
        pltpu.SemaphoreType.DMA,
        pltpu.SemaphoreType.DMA,
    ],
)
def _sc_gather(x_hbm, tgt_hbm, lse_hbm, tt_hbm, out_hbm, part_hbm,
               slab, xc, buf0, buf1, xloc, tloc, fidx, picked, lseloc,
               accv, sem0, sem1, semg):
    wid = lax.axis_index("s") * NC + lax.axis_index("c")

    base = wid * B_PER_W
    pltpu.sync_copy(x_hbm.at[pl.ds(base, B_PER_W)], xloc)
    pltpu.sync_copy(tgt_hbm.at[pl.ds(base, B_PER_W)], tloc)
    pltpu.sync_copy(lse_hbm, lseloc)

    @plsc.parallel_loop(0, B_PER_W // L, unroll=4)
    def fidx_body(k):
        xv = xloc[pl.ds(k * L, L)]
        tv = tloc[pl.ds(k * L, L)]
        fidx[pl.ds(k * L, L)] = tv * V + xv
    gather_handle = pltpu.async_copy(tt_hbm.at[fidx], picked, semg)

    def fill(buf, c):
        pltpu.sync_copy(x_hbm.at[pl.ds(c * (CT * 128), CT * 128)], xc)

        @plsc.parallel_loop(0, CT, unroll=2)
        def tile_body(t):
            for g in range(8):
                xg = xc[pl.ds(t * 128 + g * 16, L)]
                for v in range(8):
                    val = plsc.load_gather(slab, [xg + (v * V)])
                    buf[t, v, pl.ds(g * 16, L)] = val

    for s in range(SLABS):
        tr = wid + s * NW

        @pl.when(tr < TR)
        def _slab_work():
            pltpu.sync_copy(tt_hbm.at[pl.ds(tr * (8 * V), 8 * V)], slab)

            def chunk_pair(cc, carry):
                c0 = cc * 2
                for half, buf, sem in ((0, buf0, sem0), (1, buf1, sem1)):
                    c = c0 + half

                    @pl.when(cc > 0)
                    def _drain():
                        pltpu.make_async_copy(
                            buf, out_hbm.at[tr, pl.ds((c - 2) * CT, CT)],
                            sem).wait()

                    fill(buf, c)
                    pltpu.async_copy(
                        buf, out_hbm.at[tr, pl.ds(c * CT, CT)], sem)
                return carry

            lax.fori_loop(0, NCH // 2, chunk_pair, jnp.int32(0))
            pltpu.make_async_copy(
                buf0, out_hbm.at[tr, pl.ds((NCH - 2) * CT, CT)], sem0).wait()
            pltpu.make_async_copy(
                buf1, out_hbm.at[tr, pl.ds((NCH - 1) * CT, CT)], sem1).wait()

    gather_handle.wait()

    @plsc.parallel_loop(0, B_PER_W // L, unroll=4,
                        carry=jnp.zeros((L,), jnp.float32))
    def acc_loop(k, acc):
        lsev = plsc.load_gather(lseloc, [xloc[pl.ds(k * L, L)]])
        return acc + (lsev - picked[pl.ds(k * L, L)])

    acc = acc_loop
    accv[...] = acc
    pltpu.sync_copy(accv, part_hbm.at[wid])


def kernel(x, targets, table):
    xf = x.reshape(BT).astype(jnp.int32)
    tf = targets.reshape(BT).astype(jnp.int32)
    lse2, tt = _pre_call(table)
    out4, part = _sc_gather(xf, tf, lse2.reshape(V), tt.reshape(V * V))
    logits = out4.transpose(1, 3, 0, 2).reshape(BT, V)
    loss = _fin_call(part)[0, 0]
    return logits, loss


# speedup vs baseline: 3.8754x; 1.4494x over previous
import functools

import jax
import jax.numpy as jnp
from jax import lax
from jax.experimental import pallas as pl
from jax.experimental.pallas import tpu as pltpu
from jax.experimental.pallas import tpu_sc as plsc

V = 1000
BT = 4096 * 20
NC = 2
NS = 16
L = 16
NW = NC * NS
B_PER_W = BT // NW

TR = V // 8
TI = BT // 128
CT = 32
NCH = TI // CT
SLABS = 4


def _pre_body(t_ref, lse_ref, tt_ref):
    t = t_ref[...]
    m = jnp.max(t, axis=1, keepdims=True)
    s = jnp.sum(jnp.exp(t - m), axis=1, keepdims=True)
    lse_ref[...] = m + jnp.log(s)
    tt_ref[...] = t.T


_pre_call = pl.pallas_call(
    _pre_body,
    out_shape=(
        jax.ShapeDtypeStruct((V, 1), jnp.float32),
        jax.ShapeDtypeStruct((V, V), jnp.float32),
    ),
)


def _fin_body(p_ref, o_ref):
    o_ref[...] = (jnp.sum(p_ref[...]) * (1.0 / BT)).reshape(1, 1)


_fin_call = pl.pallas_call(
    _fin_body,
    out_shape=jax.ShapeDtypeStruct((1, 1), jnp.float32),
)


_mesh = plsc.VectorSubcoreMesh(core_axis_name="c", subcore_axis_name="s")


@functools.partial(
    pl.kernel,
    out_type=(
        jax.ShapeDtypeStruct((TR, TI, 8, 128), jnp.float32),
        jax.ShapeDtypeStruct((NW, L), jnp.float32),
    ),
    mesh=_mesh,
    compiler_params=pltpu.CompilerParams(
        use_tc_tiling_on_sc=False, needs_layout_passes=False),
    scratch_types=[
        pltpu.VMEM((8 * V,), jnp.float32),
        pltpu.VMEM((CT * 128,), jnp.int32),
        pltpu.VMEM((CT * 128,), jnp.int32),
        pltpu.VMEM((CT, 8, 128), jnp.float32),
        pltpu.VMEM((CT, 8, 128), jnp.float32),
        pltpu.VMEM((B_PER_W,), jnp.int32),
        pltpu.VMEM((B_PER_W,), jnp.int32),
        pltpu.VMEM((B_PER_W,), jnp.int32),
        pltpu.VMEM((B_PER_W,), jnp.float32),
        pltpu.VMEM((V,), jnp.float32),
        pltpu.VMEM((L,), jnp.float32),
        pltpu.SemaphoreType.DMA,
        pltpu.SemaphoreType.DMA,
        pltpu.SemaphoreType.DMA,
        pltpu.SemaphoreType.DMA,
        pltpu.SemaphoreType.DMA,
    ],
)
def _sc_gather(x_hbm, tgt_hbm, lse_hbm, tt_hbm, out_hbm, part_hbm,
               slab, xca, xcb, buf0, buf1, xloc, tloc, fidx, picked, lseloc,
               accv, sem0, sem1, semxa, semxb, semg):
    wid = lax.axis_index("s") * NC + lax.axis_index("c")

    base = wid * B_PER_W
    pltpu.sync_copy(x_hbm.at[pl.ds(base, B_PER_W)], xloc)
    pltpu.sync_copy(tgt_hbm.at[pl.ds(base, B_PER_W)], tloc)
    pltpu.sync_copy(lse_hbm, lseloc)

    @plsc.parallel_loop(0, B_PER_W // L, unroll=4)
    def fidx_body(k):
        xv = xloc[pl.ds(k * L, L)]
        tv = tloc[pl.ds(k * L, L)]
        fidx[pl.ds(k * L, L)] = tv * V + xv
    gather_handle = pltpu.async_copy(tt_hbm.at[fidx], picked, semg)

    def fill(buf, xc):
        @plsc.parallel_loop(0, CT * 8, unroll=4)
        def grp_body(u):
            t = u // 8
            g = u % 8
            xg = xc[pl.ds(u * L, L)]
            for v in range(8):
                val = plsc.load_gather(slab, [xg + (v * V)])
                buf[t, v, pl.ds(g * L, L)] = val

    def xchunk(c):
        return x_hbm.at[pl.ds(c * (CT * 128), CT * 128)]

    for s in range(SLABS):
        tr = wid + s * NW

        @pl.when(tr < TR)
        def _slab_work():
            pltpu.sync_copy(tt_hbm.at[pl.ds(tr * (8 * V), 8 * V)], slab)
            pltpu.sync_copy(xchunk(0), xca)
            pltpu.async_copy(xchunk(1), xcb, semxb)

            def chunk_pair(cc, carry):
                c0 = cc * 2
                for half, buf, sem, xc, semx in (
                        (0, buf0, sem0, xca, semxa),
                        (1, buf1, sem1, xcb, semxb)):
                    c = c0 + half

                    if half == 1:
                        pltpu.make_async_copy(xchunk(c), xc, semx).wait()
                    else:
                        @pl.when(cc > 0)
                        def _wait_xc():
                            pltpu.make_async_copy(xchunk(c), xc, semx).wait()

                    @pl.when(cc > 0)
                    def _drain():
                        pltpu.make_async_copy(
                            buf, out_hbm.at[tr, pl.ds((c - 2) * CT, CT)],
                            sem).wait()

                    fill(buf, xc)
                    pltpu.async_copy(
                        buf, out_hbm.at[tr, pl.ds(c * CT, CT)], sem)

                    @pl.when(c + 2 < NCH)
                    def _prefetch():
                        pltpu.async_copy(xchunk(c + 2), xc, semx)
                return carry

            lax.fori_loop(0, NCH // 2, chunk_pair, jnp.int32(0))
            pltpu.make_async_copy(
                buf0, out_hbm.at[tr, pl.ds((NCH - 2) * CT, CT)], sem0).wait()
            pltpu.make_async_copy(
                buf1, out_hbm.at[tr, pl.ds((NCH - 1) * CT, CT)], sem1).wait()

    gather_handle.wait()

    @plsc.parallel_loop(0, B_PER_W // L, unroll=4,
                        carry=jnp.zeros((L,), jnp.float32))
    def acc_loop(k, acc):
        lsev = plsc.load_gather(lseloc, [xloc[pl.ds(k * L, L)]])
        return acc + (lsev - picked[pl.ds(k * L, L)])

    acc = acc_loop
    accv[...] = acc
    pltpu.sync_copy(accv, part_hbm.at[wid])


def kernel(x, targets, table):
    xf = x.reshape(BT).astype(jnp.int32)
    tf = targets.reshape(BT).astype(jnp.int32)
    lse2, tt = _pre_call(table)
    out4, part = _sc_gather(xf, tf, lse2.reshape(V), tt.reshape(V * V))
    logits = out4.transpose(1, 3, 0, 2).reshape(BT, V)
    loss = _fin_call(part)[0, 0]
    return logits, loss


# speedup vs baseline: 4.4165x; 1.1396x over previous
import functools

import jax
import jax.numpy as jnp
from jax import lax
from jax.experimental import pallas as pl
from jax.experimental.pallas import tpu as pltpu
from jax.experimental.pallas import tpu_sc as plsc

V = 1000
BT = 4096 * 20
NC = 2
NS = 16
L = 16
NW = NC * NS
B_PER_W = BT // NW

TR = V // 8
TI = BT // 128
CT = 16
NCH = TI // CT
SLABS = 4


def _pre_body(t_ref, lse_ref, tt_ref):
    t = t_ref[...]
    m = jnp.max(t, axis=1, keepdims=True)
    s = jnp.sum(jnp.exp(t - m), axis=1, keepdims=True)
    lse_ref[...] = m + jnp.log(s)
    tt_ref[...] = t.T


_pre_call = pl.pallas_call(
    _pre_body,
    out_shape=(
        jax.ShapeDtypeStruct((V, 1), jnp.float32),
        jax.ShapeDtypeStruct((V, V), jnp.float32),
    ),
)


def _fin_body(p_ref, o_ref):
    o_ref[...] = (jnp.sum(p_ref[...]) * (1.0 / BT)).reshape(1, 1)


_fin_call = pl.pallas_call(
    _fin_body,
    out_shape=jax.ShapeDtypeStruct((1, 1), jnp.float32),
)


_mesh = plsc.VectorSubcoreMesh(core_axis_name="c", subcore_axis_name="s")


@functools.partial(
    pl.kernel,
    out_type=(
        jax.ShapeDtypeStruct((TR, TI, 8, 128), jnp.float32),
        jax.ShapeDtypeStruct((NW, L), jnp.float32),
    ),
    mesh=_mesh,
    compiler_params=pltpu.CompilerParams(
        use_tc_tiling_on_sc=False, needs_layout_passes=False),
    scratch_types=[
        pltpu.VMEM((SLABS, 8 * V), jnp.float32),
        pltpu.VMEM((CT * 128,), jnp.int32),
        pltpu.VMEM((CT * 128,), jnp.int32),
        pltpu.VMEM((SLABS, CT, 8, 128), jnp.float32),
        pltpu.VMEM((B_PER_W,), jnp.int32),
        pltpu.VMEM((B_PER_W,), jnp.int32),
        pltpu.VMEM((B_PER_W,), jnp.int32),
        pltpu.VMEM((B_PER_W,), jnp.float32),
        pltpu.VMEM((V,), jnp.float32),
        pltpu.VMEM((L,), jnp.float32),
        pltpu.SemaphoreType.DMA,
        pltpu.SemaphoreType.DMA,
        pltpu.SemaphoreType.DMA,
        pltpu.SemaphoreType.DMA,
        pltpu.SemaphoreType.DMA,
        pltpu.SemaphoreType.DMA,
        pltpu.SemaphoreType.DMA,
    ],
)
def _sc_gather(x_hbm, tgt_hbm, lse_hbm, tt_hbm, out_hbm, part_hbm,
               slab4, xca, xcb, buf4, xloc, tloc, fidx, picked, lseloc,
               accv, sem0, sem1, sem2, sem3, semxa, semxb, semg):
    wid = lax.axis_index("s") * NC + lax.axis_index("c")

    base = wid * B_PER_W
    pltpu.sync_copy(x_hbm.at[pl.ds(base, B_PER_W)], xloc)
    pltpu.sync_copy(tgt_hbm.at[pl.ds(base, B_PER_W)], tloc)
    pltpu.sync_copy(lse_hbm, lseloc)

    @plsc.parallel_loop(0, B_PER_W // L, unroll=4)
    def fidx_body(k):
        xv = xloc[pl.ds(k * L, L)]
        tv = tloc[pl.ds(k * L, L)]
        fidx[pl.ds(k * L, L)] = tv * V + xv
    gather_handle = pltpu.async_copy(tt_hbm.at[fidx], picked, semg)

    def fill(slab, buf, xc):
        @plsc.parallel_loop(0, CT * 8, unroll=4)
        def grp_body(u):
            t = u // 8
            g = u % 8
            xg = xc[pl.ds(u * L, L)]
            for v in range(8):
                val = plsc.load_gather(slab, [xg + (v * V)])
                buf[t, v, pl.ds(g * L, L)] = val

    def xchunk(c):
        return x_hbm.at[pl.ds(c * (CT * 128), CT * 128)]

    trs = [wid + s * NW for s in range(SLABS)]
    sems = (sem0, sem1, sem2, sem3)
    for s in range(SLABS):
        @pl.when(trs[s] < TR)
        def _load_slab():
            pltpu.sync_copy(tt_hbm.at[pl.ds(trs[s] * (8 * V), 8 * V)],
                            slab4.at[s])

    pltpu.sync_copy(xchunk(0), xca)
    pltpu.async_copy(xchunk(1), xcb, semxb)

    def chunk_pair(cc, carry):
        c0 = cc * 2
        for half, xc, semx in ((0, xca, semxa), (1, xcb, semxb)):
            c = c0 + half

            if half == 1:
                pltpu.make_async_copy(xchunk(c), xc, semx).wait()
            else:
                @pl.when(cc > 0)
                def _wait_xc():
                    pltpu.make_async_copy(xchunk(c), xc, semx).wait()

            for s in range(SLABS):
                tr = trs[s]

                @pl.when(tr < TR)
                def _unit():
                    @pl.when(c > 0)
                    def _drain():
                        pltpu.make_async_copy(
                            buf4.at[s],
                            out_hbm.at[tr, pl.ds((c - 1) * CT, CT)],
                            sems[s]).wait()

                    fill(slab4.at[s], buf4.at[s], xc)
                    pltpu.async_copy(
                        buf4.at[s], out_hbm.at[tr, pl.ds(c * CT, CT)],
                        sems[s])

            @pl.when(c + 2 < NCH)
            def _prefetch():
                pltpu.async_copy(xchunk(c + 2), xc, semx)
        return carry

    lax.fori_loop(0, NCH // 2, chunk_pair, jnp.int32(0))
    for s in range(SLABS):
        @pl.when(trs[s] < TR)
        def _final_drain():
            pltpu.make_async_copy(
                buf4.at[s], out_hbm.at[trs[s], pl.ds((NCH - 1) * CT, CT)],
                sems[s]).wait()

    gather_handle.wait()

    @plsc.parallel_loop(0, B_PER_W // L, unroll=4,
                        carry=jnp.zeros((L,), jnp.float32))
    def acc_loop(k, acc):
        lsev = plsc.load_gather(lseloc, [xloc[pl.ds(k * L, L)]])
        return acc + (lsev - picked[pl.ds(k * L, L)])

    acc = acc_loop
    accv[...] = acc
    pltpu.sync_copy(accv, part_hbm.at[wid])


def kernel(x, targets, table):
    xf = x.reshape(BT).astype(jnp.int32)
    tf = targets.reshape(BT).astype(jnp.int32)
    lse2, tt = _pre_call(table)
    out4, part = _sc_gather(xf, tf, lse2.reshape(V), tt.reshape(V * V))
    logits = out4.transpose(1, 3, 0, 2).reshape(BT, V)
    loss = _fin_call(part)[0, 0]
    return logits, loss
